# baseline (speedup 1.0000x reference)
import functools

import jax
import jax.numpy as jnp
from jax import lax
from jax.experimental import pallas as pl
from jax.experimental.pallas import tpu as pltpu
from jax.experimental.pallas import tpu_sc as plsc

_L = 16


def _pool_body(x_ref, o_ref, *, inv_hw):
    j = pl.program_id(1)
    part = jnp.sum(x_ref[...], axis=(1, 2)) * inv_hw

    @pl.when(j == 0)
    def _():
        o_ref[...] = part[:, None, :]

    @pl.when(j != 0)
    def _():
        o_ref[...] += part[:, None, :]


_DNUMS = lax.GatherDimensionNumbers(
    offset_dims=(), collapsed_slice_dims=(0,), start_index_map=(0,))


def _lane_splat(v, lane):
    idx = jnp.full((_L, 1), lane, jnp.int32)
    return lax.gather(v, idx, _DNUMS, (1,),
                      mode=lax.GatherScatterMode.PROMISE_IN_BOUNDS)


def _sc_scale_body(mean_hbm, wfct_hbm, wgatt_hbm, out_hbm,
                   mean_v, wfct_v, wgatt_v, s_v, *, b, c, hp):
    wid = lax.axis_index("s") * 2 + lax.axis_index("c")

    @pl.when(wid < b)
    def _():
        pltpu.sync_copy(mean_hbm.at[wid], mean_v)
        pltpu.sync_copy(wfct_hbm, wfct_v)
        pltpu.sync_copy(wgatt_hbm, wgatt_v)

        mean_regs = [mean_v[j * _L:(j + 1) * _L] for j in range(c // _L)]
        accs = [jnp.zeros((_L,), jnp.float32) for _ in range(hp // _L)]
        for ci in range(c):
            m = _lane_splat(mean_regs[ci // _L], ci % _L)
            for g in range(hp // _L):
                accs[g] += m * wfct_v[ci, g * _L:(g + 1) * _L]
        y_regs = [jnp.maximum(a, 0.0) for a in accs]

        ysp = [_lane_splat(y_regs[h // _L], h % _L) for h in range(hp)]
        for cb in range(c // _L):
            acc = jnp.zeros((_L,), jnp.float32)
            for h in range(hp):
                acc += ysp[h] * wgatt_v[h, cb * _L:(cb + 1) * _L]
            s_v[cb * _L:(cb + 1) * _L] = acc

        pltpu.sync_copy(s_v, out_hbm.at[wid])


def _mul_body(x_ref, s_ref, o_ref):
    o_ref[...] = x_ref[...] * s_ref[...][:, None, :, :]


def kernel(x, W_fc, W_gat):
    b, c, H, Wd = x.shape
    hidden = W_fc.shape[0]
    hp = ((hidden + _L - 1) // _L) * _L
    xt = jnp.transpose(x, (0, 2, 3, 1))
    HB = 32
    grid = (b, H // HB)

    means = pl.pallas_call(
        functools.partial(_pool_body, inv_hw=1.0 / (H * Wd)),
        grid=grid,
        in_specs=[pl.BlockSpec((1, HB, Wd, c), lambda i, j: (i, j, 0, 0))],
        out_specs=pl.BlockSpec((1, 1, c), lambda i, j: (i, 0, 0)),
        out_shape=jax.ShapeDtypeStruct((b, 1, c), jnp.float32),
    )(xt)

    wfct = jnp.zeros((c, hp), jnp.float32).at[:, :hidden].set(W_fc.T)
    wgatt = jnp.zeros((hp, c), jnp.float32).at[:hidden, :].set(W_gat.T)

    scale = pl.kernel(
        functools.partial(_sc_scale_body, b=b, c=c, hp=hp),
        out_type=jax.ShapeDtypeStruct((b, c), jnp.float32),
        mesh=plsc.VectorSubcoreMesh(core_axis_name="c", subcore_axis_name="s"),
        compiler_params=pltpu.CompilerParams(needs_layout_passes=False),
        scratch_types=[
            pltpu.VMEM((c,), jnp.float32),
            pltpu.VMEM((c, hp), jnp.float32),
            pltpu.VMEM((hp, c), jnp.float32),
            pltpu.VMEM((c,), jnp.float32),
        ],
    )(means.reshape(b, c), wfct, wgatt)

    out = pl.pallas_call(
        _mul_body,
        grid=grid,
        in_specs=[
            pl.BlockSpec((1, HB, Wd, c), lambda i, j: (i, j, 0, 0)),
            pl.BlockSpec((1, 1, c), lambda i, j: (i, 0, 0)),
        ],
        out_specs=pl.BlockSpec((1, HB, Wd, c), lambda i, j: (i, j, 0, 0)),
        out_shape=jax.ShapeDtypeStruct((b, H, Wd, c), jnp.float32),
    )(xt, scale.reshape(b, 1, c))

    return jnp.transpose(out, (0, 3, 1, 2))


# speedup vs baseline: 1.1035x; 1.1035x over previous
import functools

import jax
import jax.numpy as jnp
from jax import lax
from jax.experimental import pallas as pl
from jax.experimental.pallas import tpu as pltpu


def _pool_body(x_ref, o_ref, *, inv_hw):
    j = pl.program_id(1)
    part = jnp.sum(x_ref[...], axis=(1, 2)) * inv_hw

    @pl.when(j == 0)
    def _():
        o_ref[...] = part[:, None, :]

    @pl.when(j != 0)
    def _():
        o_ref[...] += part[:, None, :]


def _mul_body(x_ref, mean_ref, wfc_ref, wgat_ref, o_ref, s_ref):
    i = pl.program_id(0)
    j = pl.program_id(1)

    @pl.when(j == 0)
    def _():
        mean_i = mean_ref[i]
        y = lax.dot_general(mean_i, wfc_ref[...], (((1,), (1,)), ((), ())),
                            preferred_element_type=jnp.float32)
        y = jnp.maximum(y, 0.0)
        s = lax.dot_general(y, wgat_ref[...], (((1,), (1,)), ((), ())),
                            preferred_element_type=jnp.float32)
        s_ref[...] = s

    o_ref[...] = x_ref[...] * s_ref[...][None, :, None, :]


def kernel(x, W_fc, W_gat):
    b, c, H, Wd = x.shape
    xt = jnp.transpose(x, (0, 2, 3, 1))
    HB = 32 if H % 32 == 0 else H
    grid = (b, H // HB)

    means = pl.pallas_call(
        functools.partial(_pool_body, inv_hw=1.0 / (H * Wd)),
        grid=grid,
        in_specs=[pl.BlockSpec((1, HB, Wd, c), lambda i, j: (i, j, 0, 0))],
        out_specs=pl.BlockSpec((1, 1, c), lambda i, j: (i, 0, 0)),
        out_shape=jax.ShapeDtypeStruct((b, 1, c), jnp.float32),
    )(xt)

    out = pl.pallas_call(
        _mul_body,
        grid=grid,
        in_specs=[
            pl.BlockSpec((1, HB, Wd, c), lambda i, j: (i, j, 0, 0)),
            pl.BlockSpec((b, 1, c), lambda i, j: (0, 0, 0)),
            pl.BlockSpec(W_fc.shape, lambda i, j: (0, 0)),
            pl.BlockSpec(W_gat.shape, lambda i, j: (0, 0)),
        ],
        out_specs=pl.BlockSpec((1, HB, Wd, c), lambda i, j: (i, j, 0, 0)),
        out_shape=jax.ShapeDtypeStruct((b, H, Wd, c), jnp.float32),
        scratch_shapes=[pltpu.VMEM((1, c), jnp.float32)],
    )(xt, means, W_fc, W_gat)

    return jnp.transpose(out, (0, 3, 1, 2))
